# baseline (speedup 1.0000x reference)
import jax
import jax.numpy as jnp
import numpy as np
from jax import lax
from jax.experimental import pallas as pl
from jax.experimental.pallas import tpu as pltpu
from jax.experimental.pallas import tpu_sc as plsc

NEG_SLOPE = 0.2
NC = 2
NS = 16
L = 16

N = 10000
E = 320000
H = 8
F = 32
HF = H * F
HHF = HF // 2

NPAD = 10240
TILE_N = NPAD // NS
ZROWS = 128

K1 = 400
EPT1 = E // (NC * NS)
NCHUNK1 = EPT1 // K1

K2 = 200
EPT2 = E // NS
NCHUNK2 = EPT2 // K2

def _take16(x, idx):
    dn = lax.GatherDimensionNumbers(
        offset_dims=(), collapsed_slice_dims=(0,), start_index_map=(0,))
    return lax.gather(x, idx[:, None], dn, (1,),
                      mode=lax.GatherScatterMode.PROMISE_IN_BOUNDS)


def _proj_body(feat_ref, w_ref, bs_ref, bd_ref,
               fta_ref, ftb_ref, esx_ref, edcx_ref, esa_ref, eda_ref):
    i = pl.program_id(0)
    blk = feat_ref.shape[0]
    ft = jnp.dot(feat_ref[...], w_ref[...], preferred_element_type=jnp.float32)
    fta_ref[...] = ft[:, :HHF]
    ftb_ref[...] = ft[:, HHF:]
    esa_ref[pl.ds(i * blk, blk), :] = jnp.dot(
        ft, bs_ref[...], preferred_element_type=jnp.float32)
    eda_ref[pl.ds(i * blk, blk), :] = jnp.dot(
        ft, bd_ref[...], preferred_element_type=jnp.float32)

    @pl.when(i == pl.num_programs(0) - 1)
    def _():
        es = esa_ref[...]
        ed = eda_ref[...]
        m = jnp.max(es, axis=0, keepdims=True)
        cc = m + ed
        cc = jnp.where(cc > 0, cc, NEG_SLOPE * cc)
        esx_ref[...] = jnp.concatenate([es, jnp.zeros_like(es)], axis=1)
        edcx_ref[...] = jnp.concatenate([ed, cc], axis=1)


def _pass1_body(esx_hbm, edcx_hbm, src_hbm, dst_hbm,
                wd_hbm, s0_hbm, s1_hbm,
                src_v, dst_v, gs, gd, wbuf, sacc):
    c = lax.axis_index("c")
    s = lax.axis_index("s")
    wid = s * NC + c
    lane = lax.iota(jnp.int32, L)
    idx_c = (lane & 7) + 8

    @pl.loop(0, ZROWS)
    def _(i):
        wbuf[i] = jnp.zeros((L,), jnp.float32)

    @pl.loop(0, TILE_N // ZROWS)
    def _(i):
        pltpu.sync_copy(wbuf.at[pl.ds(0, ZROWS)],
                        sacc.at[pl.ds(s * TILE_N + i * ZROWS, ZROWS)])

    plsc.subcore_barrier()

    base = wid * EPT1

    @pl.loop(0, NCHUNK1)
    def _(t):
        off = base + t * K1
        pltpu.sync_copy(src_hbm.at[pl.ds(off, K1)], src_v)
        pltpu.sync_copy(dst_hbm.at[pl.ds(off, K1)], dst_v)
        pltpu.sync_copy(esx_hbm.at[src_v], gs)
        pltpu.sync_copy(edcx_hbm.at[dst_v], gd)

        @pl.loop(0, K1 // 8)
        def _(g):
            for jj in range(8):
                j = g * 8 + jj
                u = gs[j] + gd[j]
                lr = jnp.where(u > 0, u, NEG_SLOPE * u)
                cvec = _take16(u, idx_c)
                wbuf[j] = jnp.exp(lr - cvec)

        pltpu.sync_copy(wbuf, sacc.at[dst_v], add=True)
        pltpu.sync_copy(wbuf, wd_hbm.at[pl.ds(off, K1)])

    plsc.subcore_barrier()
    rs = pl.ds(s * TILE_N, TILE_N)

    @pl.when(c == 0)
    def _():
        pltpu.sync_copy(sacc.at[rs], s0_hbm.at[rs])

    @pl.when(c == 1)
    def _():
        pltpu.sync_copy(sacc.at[rs], s1_hbm.at[rs])


def _pass2_body(fta_hbm, ftb_hbm, wd_hbm, src_hbm, dst_hbm, dist_hbm,
                numa_hbm, numb_hbm,
                src_v, dst_v, dist_v, wdc, msg, acc):
    c = lax.axis_index("c")
    s = lax.axis_index("s")

    @pl.loop(0, ZROWS)
    def _(i):
        for l8 in range(8):
            msg[i, pl.ds(l8 * 16, 16)] = jnp.zeros((L,), jnp.float32)

    @pl.loop(0, TILE_N // ZROWS)
    def _(i):
        pltpu.sync_copy(msg.at[pl.ds(0, ZROWS)],
                        acc.at[pl.ds(s * TILE_N + i * ZROWS, ZROWS)])

    plsc.subcore_barrier()

    hidx = [jnp.full((L,), 0, jnp.int32) + (c * 4 + hh) for hh in range(4)]
    base = s * EPT2

    @pl.loop(0, NCHUNK2)
    def _(t):
        off = base + t * K2
        pltpu.sync_copy(src_hbm.at[pl.ds(off, K2)], src_v)
        pltpu.sync_copy(dst_hbm.at[pl.ds(off, K2)], dst_v)
        pltpu.sync_copy(dist_hbm.at[pl.ds(off, K2)], dist_v)
        pltpu.sync_copy(wd_hbm.at[pl.ds(off, K2)], wdc)

        @pl.when(c == 0)
        def _():
            pltpu.sync_copy(fta_hbm.at[src_v], msg)

        @pl.when(c == 1)
        def _():
            pltpu.sync_copy(ftb_hbm.at[src_v], msg)

        @pl.loop(0, K2 // 16)
        def _(g):
            dreg = dist_v[pl.ds(g * 16, 16)]
            for jj in range(16):
                j = g * 16 + jj
                wrow = wdc[j] * _take16(dreg, jnp.full((L,), jj, jnp.int32))
                for hh in range(4):
                    coef = _take16(wrow, hidx[hh])
                    for half in range(2):
                        sl = pl.ds(hh * 32 + half * 16, 16)
                        msg[j, sl] = msg[j, sl] * coef

        pltpu.sync_copy(msg, acc.at[dst_v], add=True)

    plsc.subcore_barrier()
    rs = pl.ds(s * TILE_N, TILE_N)

    @pl.when(c == 0)
    def _():
        pltpu.sync_copy(acc.at[rs], numa_hbm.at[rs])

    @pl.when(c == 1)
    def _():
        pltpu.sync_copy(acc.at[rs], numb_hbm.at[rs])


def _final_body(numa_ref, numb_ref, s0_ref, s1_ref, r4_ref, out_ref):
    s8 = s0_ref[...][:, :H] + s1_ref[...][:, :H]
    r4 = r4_ref[...]
    sa = jnp.dot(s8[:, 0:4], r4, preferred_element_type=jnp.float32)
    sb = jnp.dot(s8[:, 4:8], r4, preferred_element_type=jnp.float32)
    outa = jnp.where(sa > 0, numa_ref[...] / sa, 0.0)
    outb = jnp.where(sb > 0, numb_ref[...] / sb, 0.0)
    out_ref[...] = jnp.concatenate([outa, outb], axis=1)


def kernel(feat, dist, edge_index, W, w_att_src, w_att_dst):
    n, d = feat.shape
    srcs = edge_index[0].astype(jnp.int32)
    dsts = edge_index[1].astype(jnp.int32)
    distf = dist.reshape(E).astype(jnp.float32)

    eye = jnp.eye(H, dtype=jnp.float32)
    bs = (w_att_src[0][:, :, None] * eye[:, None, :]).reshape(HF, H)
    bd = (w_att_dst[0][:, :, None] * eye[:, None, :]).reshape(HF, H)

    blk = 1000
    fta, ftb, esx, edcx = pl.pallas_call(
        _proj_body,
        grid=(n // blk,),
        in_specs=[
            pl.BlockSpec((blk, d), lambda i: (i, 0)),
            pl.BlockSpec((d, HF), lambda i: (0, 0)),
            pl.BlockSpec((HF, H), lambda i: (0, 0)),
            pl.BlockSpec((HF, H), lambda i: (0, 0)),
        ],
        out_specs=[
            pl.BlockSpec((blk, HHF), lambda i: (i, 0)),
            pl.BlockSpec((blk, HHF), lambda i: (i, 0)),
            pl.BlockSpec((N, 2 * H), lambda i: (0, 0)),
            pl.BlockSpec((N, 2 * H), lambda i: (0, 0)),
        ],
        out_shape=[
            jax.ShapeDtypeStruct((N, HHF), jnp.float32),
            jax.ShapeDtypeStruct((N, HHF), jnp.float32),
            jax.ShapeDtypeStruct((N, 2 * H), jnp.float32),
            jax.ShapeDtypeStruct((N, 2 * H), jnp.float32),
        ],
        scratch_shapes=[
            pltpu.VMEM((N, H), jnp.float32),
            pltpu.VMEM((N, H), jnp.float32),
        ],
    )(feat, W, bs, bd)

    mesh = plsc.VectorSubcoreMesh(core_axis_name="c", subcore_axis_name="s")
    sc_params = pltpu.CompilerParams(use_tc_tiling_on_sc=False)

    wd, s0, s1 = pl.kernel(
        _pass1_body,
        compiler_params=sc_params,
        out_type=[
            jax.ShapeDtypeStruct((E, 16), jnp.float32),
            jax.ShapeDtypeStruct((NPAD, 16), jnp.float32),
            jax.ShapeDtypeStruct((NPAD, 16), jnp.float32),
        ],
        mesh=mesh,
        scratch_types=[
            pltpu.VMEM((K1,), jnp.int32),
            pltpu.VMEM((K1,), jnp.int32),
            pltpu.VMEM((K1, 16), jnp.float32),
            pltpu.VMEM((K1, 16), jnp.float32),
            pltpu.VMEM((K1, 16), jnp.float32),
            pltpu.VMEM_SHARED((NPAD, 16), jnp.float32),
        ],
    )(esx, edcx, srcs, dsts)

    numa, numb = pl.kernel(
        _pass2_body,
        compiler_params=sc_params,
        out_type=[
            jax.ShapeDtypeStruct((NPAD, HHF), jnp.float32),
            jax.ShapeDtypeStruct((NPAD, HHF), jnp.float32),
        ],
        mesh=mesh,
        scratch_types=[
            pltpu.VMEM((K2,), jnp.int32),
            pltpu.VMEM((K2,), jnp.int32),
            pltpu.VMEM((K2,), jnp.float32),
            pltpu.VMEM((K2, 16), jnp.float32),
            pltpu.VMEM((K2, HHF), jnp.float32),
            pltpu.VMEM_SHARED((NPAD, HHF), jnp.float32),
        ],
    )(fta, ftb, wd, srcs, dsts, distf)

    r4 = jnp.kron(jnp.eye(4, dtype=jnp.float32), jnp.ones((1, F), jnp.float32))

    out = pl.pallas_call(
        _final_body,
        grid=(n // blk,),
        in_specs=[
            pl.BlockSpec((blk, HHF), lambda i: (i, 0)),
            pl.BlockSpec((blk, HHF), lambda i: (i, 0)),
            pl.BlockSpec((blk, 16), lambda i: (i, 0)),
            pl.BlockSpec((blk, 16), lambda i: (i, 0)),
            pl.BlockSpec((4, HHF), lambda i: (0, 0)),
        ],
        out_specs=pl.BlockSpec((blk, HF), lambda i: (i, 0)),
        out_shape=jax.ShapeDtypeStruct((N, HF), jnp.float32),
    )(numa, numb, s0, s1, r4)

    return out


# speedup vs baseline: 1.0252x; 1.0252x over previous
import jax
import jax.numpy as jnp
import numpy as np
from jax import lax
from jax.experimental import pallas as pl
from jax.experimental.pallas import tpu as pltpu
from jax.experimental.pallas import tpu_sc as plsc

NEG_SLOPE = 0.2
NC = 2
NS = 16
L = 16

N = 10000
E = 320000
H = 8
F = 32
HF = H * F
HHF = HF // 2

NPAD = 10240
TILE_N = NPAD // NS
ZROWS = 128

K1 = 400
EPT1 = E // (NC * NS)
NCHUNK1 = EPT1 // K1

K2 = 200
EPT2 = E // NS
NCHUNK2 = EPT2 // K2

def _take16(x, idx):
    dn = lax.GatherDimensionNumbers(
        offset_dims=(), collapsed_slice_dims=(0,), start_index_map=(0,))
    return lax.gather(x, idx[:, None], dn, (1,),
                      mode=lax.GatherScatterMode.PROMISE_IN_BOUNDS)


def _proj_body(feat_ref, w_ref, bs_ref, bd_ref,
               fta_ref, ftb_ref, esx_ref, edcx_ref, esa_ref, eda_ref):
    i = pl.program_id(0)
    blk = feat_ref.shape[0]
    ft = jnp.dot(feat_ref[...], w_ref[...], preferred_element_type=jnp.float32)
    fta_ref[...] = ft[:, :HHF]
    ftb_ref[...] = ft[:, HHF:]
    esa_ref[pl.ds(i * blk, blk), :] = jnp.dot(
        ft, bs_ref[...], preferred_element_type=jnp.float32)
    eda_ref[pl.ds(i * blk, blk), :] = jnp.dot(
        ft, bd_ref[...], preferred_element_type=jnp.float32)

    @pl.when(i == pl.num_programs(0) - 1)
    def _():
        es = esa_ref[...]
        ed = eda_ref[...]
        m = jnp.max(es, axis=0, keepdims=True)
        cc = m + ed
        cc = jnp.where(cc > 0, cc, NEG_SLOPE * cc)
        esx_ref[...] = jnp.concatenate([es, jnp.zeros_like(es)], axis=1)
        edcx_ref[...] = jnp.concatenate([ed, cc], axis=1)


def _pass1_body(esx_hbm, edcx_hbm, src_hbm, dst_hbm, dist_hbm,
                wd_hbm, s0_hbm, s1_hbm,
                src_v, dst_v, dist_v, gs, gd, wbuf, wdbuf, sacc):
    c = lax.axis_index("c")
    s = lax.axis_index("s")
    wid = s * NC + c
    lane = lax.iota(jnp.int32, L)
    idx_c = (lane & 7) + 8
    mask_lo = lane < 8

    @pl.loop(0, ZROWS)
    def _(i):
        wbuf[i] = jnp.zeros((L,), jnp.float32)

    @pl.loop(0, TILE_N // ZROWS)
    def _(i):
        pltpu.sync_copy(wbuf.at[pl.ds(0, ZROWS)],
                        sacc.at[pl.ds(s * TILE_N + i * ZROWS, ZROWS)])

    plsc.subcore_barrier()

    base = wid * EPT1

    @pl.loop(0, NCHUNK1)
    def _(t):
        off = base + t * K1
        pltpu.sync_copy(src_hbm.at[pl.ds(off, K1)], src_v)
        pltpu.sync_copy(dst_hbm.at[pl.ds(off, K1)], dst_v)
        pltpu.sync_copy(dist_hbm.at[pl.ds(off, K1)], dist_v)
        pltpu.sync_copy(esx_hbm.at[src_v], gs)
        pltpu.sync_copy(edcx_hbm.at[dst_v], gd)

        @pl.loop(0, K1 // 16)
        def _(g):
            dreg = dist_v[pl.ds(g * 16, 16)]
            for jj in range(16):
                j = g * 16 + jj
                u = gs[j] + gd[j]
                lr = jnp.where(u > 0, u, NEG_SLOPE * u)
                cvec = _take16(u, idx_c)
                w = jnp.exp(lr - cvec)
                w = jnp.where(mask_lo, w, 0.0)
                wbuf[j] = w
                wdbuf[j] = w * _take16(dreg, jnp.full((L,), jj, jnp.int32))

        pltpu.sync_copy(wbuf, sacc.at[dst_v], add=True)
        pltpu.sync_copy(wdbuf, wd_hbm.at[pl.ds(off, K1)])

    plsc.subcore_barrier()
    rs = pl.ds(s * TILE_N, TILE_N)

    @pl.when(c == 0)
    def _():
        pltpu.sync_copy(sacc.at[rs], s0_hbm.at[rs])

    @pl.when(c == 1)
    def _():
        pltpu.sync_copy(sacc.at[rs], s1_hbm.at[rs])


def _pass2_body(fta_hbm, ftb_hbm, wd_hbm, src_hbm, dst_hbm,
                numa_hbm, numb_hbm,
                src_v, dst_v, wdc, msg, acc):
    c = lax.axis_index("c")
    s = lax.axis_index("s")

    @pl.loop(0, ZROWS)
    def _(i):
        for l8 in range(8):
            msg[i, pl.ds(l8 * 16, 16)] = jnp.zeros((L,), jnp.float32)

    @pl.loop(0, TILE_N // ZROWS)
    def _(i):
        pltpu.sync_copy(msg.at[pl.ds(0, ZROWS)],
                        acc.at[pl.ds(s * TILE_N + i * ZROWS, ZROWS)])

    plsc.subcore_barrier()

    hidx = [jnp.full((L,), 0, jnp.int32) + (c * 4 + hh) for hh in range(4)]
    base = s * EPT2

    @pl.loop(0, NCHUNK2)
    def _(t):
        off = base + t * K2
        pltpu.sync_copy(src_hbm.at[pl.ds(off, K2)], src_v)
        pltpu.sync_copy(dst_hbm.at[pl.ds(off, K2)], dst_v)
        pltpu.sync_copy(wd_hbm.at[pl.ds(off, K2)], wdc)

        @pl.when(c == 0)
        def _():
            pltpu.sync_copy(fta_hbm.at[src_v], msg)

        @pl.when(c == 1)
        def _():
            pltpu.sync_copy(ftb_hbm.at[src_v], msg)

        @pl.loop(0, K2)
        def _(j):
            wrow = wdc[j]
            for hh in range(4):
                coef = _take16(wrow, hidx[hh])
                for half in range(2):
                    sl = pl.ds(hh * 32 + half * 16, 16)
                    msg[j, sl] = msg[j, sl] * coef

        pltpu.sync_copy(msg, acc.at[dst_v], add=True)

    plsc.subcore_barrier()
    rs = pl.ds(s * TILE_N, TILE_N)

    @pl.when(c == 0)
    def _():
        pltpu.sync_copy(acc.at[rs], numa_hbm.at[rs])

    @pl.when(c == 1)
    def _():
        pltpu.sync_copy(acc.at[rs], numb_hbm.at[rs])


def _final_body(numa_ref, numb_ref, s0_ref, s1_ref, r4_ref, out_ref):
    s8 = s0_ref[...][:, :H] + s1_ref[...][:, :H]
    r4 = r4_ref[...]
    sa = jnp.dot(s8[:, 0:4], r4, preferred_element_type=jnp.float32)
    sb = jnp.dot(s8[:, 4:8], r4, preferred_element_type=jnp.float32)
    outa = jnp.where(sa > 0, numa_ref[...] / sa, 0.0)
    outb = jnp.where(sb > 0, numb_ref[...] / sb, 0.0)
    out_ref[...] = jnp.concatenate([outa, outb], axis=1)


def kernel(feat, dist, edge_index, W, w_att_src, w_att_dst):
    n, d = feat.shape
    srcs = edge_index[0].astype(jnp.int32)
    dsts = edge_index[1].astype(jnp.int32)
    distf = dist.reshape(E).astype(jnp.float32)

    eye = jnp.eye(H, dtype=jnp.float32)
    bs = (w_att_src[0][:, :, None] * eye[:, None, :]).reshape(HF, H)
    bd = (w_att_dst[0][:, :, None] * eye[:, None, :]).reshape(HF, H)

    blk = 1000
    fta, ftb, esx, edcx = pl.pallas_call(
        _proj_body,
        grid=(n // blk,),
        in_specs=[
            pl.BlockSpec((blk, d), lambda i: (i, 0)),
            pl.BlockSpec((d, HF), lambda i: (0, 0)),
            pl.BlockSpec((HF, H), lambda i: (0, 0)),
            pl.BlockSpec((HF, H), lambda i: (0, 0)),
        ],
        out_specs=[
            pl.BlockSpec((blk, HHF), lambda i: (i, 0)),
            pl.BlockSpec((blk, HHF), lambda i: (i, 0)),
            pl.BlockSpec((N, 2 * H), lambda i: (0, 0)),
            pl.BlockSpec((N, 2 * H), lambda i: (0, 0)),
        ],
        out_shape=[
            jax.ShapeDtypeStruct((N, HHF), jnp.float32),
            jax.ShapeDtypeStruct((N, HHF), jnp.float32),
            jax.ShapeDtypeStruct((N, 2 * H), jnp.float32),
            jax.ShapeDtypeStruct((N, 2 * H), jnp.float32),
        ],
        scratch_shapes=[
            pltpu.VMEM((N, H), jnp.float32),
            pltpu.VMEM((N, H), jnp.float32),
        ],
    )(feat, W, bs, bd)

    mesh = plsc.VectorSubcoreMesh(core_axis_name="c", subcore_axis_name="s")
    sc_params = pltpu.CompilerParams(use_tc_tiling_on_sc=False)

    wd, s0, s1 = pl.kernel(
        _pass1_body,
        compiler_params=sc_params,
        out_type=[
            jax.ShapeDtypeStruct((E, 16), jnp.float32),
            jax.ShapeDtypeStruct((NPAD, 16), jnp.float32),
            jax.ShapeDtypeStruct((NPAD, 16), jnp.float32),
        ],
        mesh=mesh,
        scratch_types=[
            pltpu.VMEM((K1,), jnp.int32),
            pltpu.VMEM((K1,), jnp.int32),
            pltpu.VMEM((K1,), jnp.float32),
            pltpu.VMEM((K1, 16), jnp.float32),
            pltpu.VMEM((K1, 16), jnp.float32),
            pltpu.VMEM((K1, 16), jnp.float32),
            pltpu.VMEM((K1, 16), jnp.float32),
            pltpu.VMEM_SHARED((NPAD, 16), jnp.float32),
        ],
    )(esx, edcx, srcs, dsts, distf)

    numa, numb = pl.kernel(
        _pass2_body,
        compiler_params=sc_params,
        out_type=[
            jax.ShapeDtypeStruct((NPAD, HHF), jnp.float32),
            jax.ShapeDtypeStruct((NPAD, HHF), jnp.float32),
        ],
        mesh=mesh,
        scratch_types=[
            pltpu.VMEM((K2,), jnp.int32),
            pltpu.VMEM((K2,), jnp.int32),
            pltpu.VMEM((K2, 16), jnp.float32),
            pltpu.VMEM((K2, HHF), jnp.float32),
            pltpu.VMEM_SHARED((NPAD, HHF), jnp.float32),
        ],
    )(fta, ftb, wd, srcs, dsts)

    r4 = jnp.kron(jnp.eye(4, dtype=jnp.float32), jnp.ones((1, F), jnp.float32))

    out = pl.pallas_call(
        _final_body,
        grid=(n // blk,),
        in_specs=[
            pl.BlockSpec((blk, HHF), lambda i: (i, 0)),
            pl.BlockSpec((blk, HHF), lambda i: (i, 0)),
            pl.BlockSpec((blk, 16), lambda i: (i, 0)),
            pl.BlockSpec((blk, 16), lambda i: (i, 0)),
            pl.BlockSpec((4, HHF), lambda i: (0, 0)),
        ],
        out_specs=pl.BlockSpec((blk, HF), lambda i: (i, 0)),
        out_shape=jax.ShapeDtypeStruct((N, HF), jnp.float32),
    )(numa, numb, s0, s1, r4)

    return out
